# baseline (speedup 1.0000x reference)
import functools

import jax
import jax.numpy as jnp
from jax import lax
from jax.experimental import pallas as pl
from jax.experimental.pallas import tpu as pltpu
from jax.experimental.pallas import tpu_sc as plsc

N = 10000
D = 128
E = 320000

NC = 2
NS = 16
NW = NC * NS

K = 128
CH0 = 80
CH1 = 80
PG = 16
TOT_CHUNKS = NS * (CH0 + CH1)
EPAD = TOT_CHUNKS * K
CORE1_ROW = NS * CH0

ROWS_PER_TILE = 640
NSH = ROWS_PER_TILE * NS
CROWS = NSH // 16
CSTRIPE = CROWS // NS


def _linear_body(x_ref, w_ref, b_ref, y_ref):
    y_ref[...] = lax.dot_general(
        x_ref[...], w_ref[...], (((1,), (1,)), ((), ())),
        preferred_element_type=jnp.float32) + b_ref[...]


def _linear(x, w, b2):
    blk = 1000
    return pl.pallas_call(
        _linear_body,
        grid=(N // blk,),
        in_specs=[
            pl.BlockSpec((blk, D), lambda i: (i, 0)),
            pl.BlockSpec((D, D), lambda i: (0, 0)),
            pl.BlockSpec((1, D), lambda i: (0, 0)),
        ],
        out_specs=pl.BlockSpec((blk, D), lambda i: (i, 0)),
        out_shape=jax.ShapeDtypeStruct((N, D), jnp.float32),
    )(x, w, b2)


def _sc_body(y_hbm, src_hbm, dst_hbm, zrow_hbm, zcnt_hbm, iota_hbm,
             agg_out, cnt_out,
             src_pg, dst_pg, rows_a, cnt_loc, iota_v, agg_sh, cnt_sh,
             gsem_a, csem):
    c = lax.axis_index("c")
    s = lax.axis_index("s")
    pltpu.sync_copy(zrow_hbm, agg_sh.at[pl.ds(s * ROWS_PER_TILE, ROWS_PER_TILE)])
    pltpu.sync_copy(zcnt_hbm.at[pl.ds(0, CSTRIPE)],
                    cnt_sh.at[pl.ds(s * CSTRIPE, CSTRIPE)])
    pltpu.sync_copy(zcnt_hbm, cnt_loc)
    pltpu.sync_copy(iota_hbm, iota_v)
    plsc.subcore_barrier()

    ones16 = jnp.ones((16,), jnp.float32)

    row0 = jnp.where(c == 0, s * CH0, CORE1_ROW + s * CH1)
    npages = jnp.where(c == 0, CH0 // PG, CH1 // PG)

    def count(j):
        def cbody(i, carry):
            d = dst_pg[j, pl.ds(i * 16, 16)]
            r = lax.shift_right_logical(d, 4)
            col = lax.bitwise_and(d, 15)
            plsc.addupdate_scatter(cnt_loc, [r, col], ones16)
            return carry

        lax.fori_loop(0, K // 16, cbody, 0)

    def page(p, carry):
        base = row0 + p * PG
        pltpu.sync_copy(src_hbm.at[pl.ds(base, PG)], src_pg)
        pltpu.sync_copy(dst_hbm.at[pl.ds(base, PG)], dst_pg)

        def chunk(j, carry2):
            desc = pltpu.async_copy(y_hbm.at[src_pg.at[j]], rows_a, gsem_a)
            count(j)
            desc.wait()
            pltpu.sync_copy(rows_a, agg_sh.at[dst_pg.at[j]], add=True)
            return carry2

        lax.fori_loop(0, PG, chunk, 0)
        return carry

    lax.fori_loop(0, npages, page, 0)

    for t in range(CROWS // 128):
        pltpu.async_copy(cnt_loc.at[pl.ds(t * 128, 128)],
                         cnt_sh.at[iota_v.at[t]], csem, add=True)
    for t in range(CROWS // 128):
        pltpu.make_async_copy(cnt_loc.at[pl.ds(t * 128, 128)],
                              cnt_sh.at[iota_v.at[t]], csem).wait()
    plsc.subcore_barrier()

    pltpu.sync_copy(agg_sh.at[pl.ds(s * ROWS_PER_TILE, ROWS_PER_TILE)],
                    agg_out.at[c, pl.ds(s * ROWS_PER_TILE, ROWS_PER_TILE)])
    pltpu.sync_copy(cnt_sh.at[pl.ds(s * CSTRIPE, CSTRIPE)],
                    cnt_out.at[c, pl.ds(s * CSTRIPE, CSTRIPE)])


def _sc_scatter(y, src_r, dst_r, zrow, zcnt, iota_r):
    mesh = plsc.VectorSubcoreMesh(core_axis_name="c", subcore_axis_name="s")
    fn = pl.kernel(
        _sc_body,
        out_type=(
            jax.ShapeDtypeStruct((NC, NSH, D), jnp.float32),
            jax.ShapeDtypeStruct((NC, CROWS, 16), jnp.float32),
        ),
        mesh=mesh,
        scratch_types=[
            pltpu.VMEM((PG, K), jnp.int32),
            pltpu.VMEM((PG, K), jnp.int32),
            pltpu.VMEM((K, D), jnp.float32),
            pltpu.VMEM((CROWS, 16), jnp.float32),
            pltpu.VMEM((CROWS // 128, 128), jnp.int32),
            pltpu.VMEM_SHARED((NSH, D), jnp.float32),
            pltpu.VMEM_SHARED((CROWS, 16), jnp.float32),
            pltpu.SemaphoreType.DMA,
            pltpu.SemaphoreType.DMA,
        ],
        compiler_params=pltpu.CompilerParams(use_tc_tiling_on_sc=False,
                                             needs_layout_passes=False),
    )
    return fn(y, src_r, dst_r, zrow, zcnt, iota_r)


def _gru_body(y_ref, part_ref, cnt_ref, wih_ref, whh_ref, bih_ref, bhh_ref,
              out_ref):
    y = y_ref[...]
    agg = part_ref[0] + part_ref[1] + y
    cnt = cnt_ref[0] + cnt_ref[1] + 1.0
    h = agg / cnt

    w_ih = wih_ref[...]
    w_hh = whh_ref[...]
    b_ih = bih_ref[...]
    b_hh = bhh_ref[...]

    dn = (((1,), (1,)), ((), ()))
    gi1 = lax.dot_general(h, w_ih, dn, preferred_element_type=jnp.float32) + b_ih
    z1 = jax.nn.sigmoid(gi1[:, 128:256] + b_hh[:, 128:256])
    r1 = jax.nn.sigmoid(gi1[:, 0:128] + b_hh[:, 0:128])
    n1 = jnp.tanh(gi1[:, 256:384] + r1 * b_hh[:, 256:384])
    h1 = (1.0 - z1) * n1
    gi2 = lax.dot_general(y, w_ih, dn, preferred_element_type=jnp.float32) + b_ih
    gh2 = lax.dot_general(h1, w_hh, dn, preferred_element_type=jnp.float32) + b_hh
    r2 = jax.nn.sigmoid(gi2[:, 0:128] + gh2[:, 0:128])
    z2 = jax.nn.sigmoid(gi2[:, 128:256] + gh2[:, 128:256])
    n2 = jnp.tanh(gi2[:, 256:384] + r2 * gh2[:, 256:384])
    out_ref[...] = (1.0 - z2) * n2 + z2 * h1


def _mean_gru(y, part, cntp, w_ih, w_hh, b_ih2, b_hh2):
    blk = 1024
    grid = -(-N // blk)
    return pl.pallas_call(
        _gru_body,
        grid=(grid,),
        in_specs=[
            pl.BlockSpec((blk, D), lambda i: (i, 0)),
            pl.BlockSpec((NC, blk, D), lambda i: (0, i, 0)),
            pl.BlockSpec((NC, blk, 1), lambda i: (0, i, 0)),
            pl.BlockSpec((3 * D, D), lambda i: (0, 0)),
            pl.BlockSpec((3 * D, D), lambda i: (0, 0)),
            pl.BlockSpec((1, 3 * D), lambda i: (0, 0)),
            pl.BlockSpec((1, 3 * D), lambda i: (0, 0)),
        ],
        out_specs=pl.BlockSpec((blk, D), lambda i: (i, 0)),
        out_shape=jax.ShapeDtypeStruct((N, D), jnp.float32),
    )(y, part, cntp, w_ih, w_hh, b_ih2, b_hh2)


def kernel(x, edge_index, W, b, W_ih, W_hh, b_ih, b_hh):
    src = edge_index[0].astype(jnp.int32)
    dst = edge_index[1].astype(jnp.int32)
    pad = EPAD - E
    src_r = jnp.concatenate([src, jnp.zeros((pad,), jnp.int32)]).reshape(TOT_CHUNKS, K)
    pad_dst = N + (jnp.arange(pad, dtype=jnp.int32) % (NSH - N))
    dst_r = jnp.concatenate([dst, pad_dst]).reshape(TOT_CHUNKS, K)

    zrow = jnp.zeros((ROWS_PER_TILE, D), jnp.float32)
    zcnt = jnp.zeros((CROWS, 16), jnp.float32)
    iota_r = jnp.arange(CROWS, dtype=jnp.int32).reshape(CROWS // 128, 128)

    y = _linear(x, W, b.reshape(1, D))
    part, cntp = _sc_scatter(y, src_r, dst_r, zrow, zcnt, iota_r)
    cntp = cntp.reshape(NC, NSH, 1)
    return _mean_gru(y, part, cntp, W_ih, W_hh,
                     b_ih.reshape(1, 3 * D), b_hh.reshape(1, 3 * D))


# speedup vs baseline: 2.0551x; 2.0551x over previous
import functools

import jax
import jax.numpy as jnp
from jax import lax
from jax.experimental import pallas as pl
from jax.experimental.pallas import tpu as pltpu
from jax.experimental.pallas import tpu_sc as plsc

N = 10000
D = 128
E = 320000

NC = 2
NS = 16
NW = NC * NS

K = 128
CH0 = 80
CH1 = 80
PG = 16
TOT_CHUNKS = NS * (CH0 + CH1)
EPAD = TOT_CHUNKS * K
CORE1_ROW = NS * CH0

ROWS_PER_TILE = 640
NSH = ROWS_PER_TILE * NS
CROWS = NSH // 16
CSTRIPE = CROWS // NS


def _linear_body(x_ref, w_ref, b_ref, y_ref):
    y_ref[...] = lax.dot_general(
        x_ref[...], w_ref[...], (((1,), (1,)), ((), ())),
        preferred_element_type=jnp.float32) + b_ref[...]


def _linear(x, w, b2):
    blk = 1000
    return pl.pallas_call(
        _linear_body,
        grid=(N // blk,),
        in_specs=[
            pl.BlockSpec((blk, D), lambda i: (i, 0)),
            pl.BlockSpec((D, D), lambda i: (0, 0)),
            pl.BlockSpec((1, D), lambda i: (0, 0)),
        ],
        out_specs=pl.BlockSpec((blk, D), lambda i: (i, 0)),
        out_shape=jax.ShapeDtypeStruct((N, D), jnp.float32),
    )(x, w, b2)


def _sc_body(y_hbm, src_hbm, dst_hbm, zrow_hbm, zcnt_hbm, iota_hbm,
             agg_out, cnt_out,
             src_pg, dst_pg, rows_a, cnt_loc, iota_v, agg_sh, cnt_sh,
             gsem_a, csem):
    c = lax.axis_index("c")
    s = lax.axis_index("s")
    pltpu.sync_copy(zrow_hbm, agg_sh.at[pl.ds(s * ROWS_PER_TILE, ROWS_PER_TILE)])
    pltpu.sync_copy(zcnt_hbm.at[pl.ds(0, CSTRIPE)],
                    cnt_sh.at[pl.ds(s * CSTRIPE, CSTRIPE)])
    pltpu.sync_copy(zcnt_hbm, cnt_loc)
    pltpu.sync_copy(iota_hbm, iota_v)
    plsc.subcore_barrier()

    ones16 = jnp.ones((16,), jnp.float32)

    row0 = jnp.where(c == 0, s * CH0, CORE1_ROW + s * CH1)
    npages = jnp.where(c == 0, CH0 // PG, CH1 // PG)

    def count(j):
        def cbody(i, carry):
            d = dst_pg[j, pl.ds(i * 16, 16)]
            r = lax.shift_right_logical(d, 4)
            col = lax.bitwise_and(d, 15)
            plsc.addupdate_scatter(cnt_loc, [r, col], ones16)
            return carry

        lax.fori_loop(0, K // 16, cbody, 0)

    def page(p, carry):
        base = row0 + p * PG
        pltpu.sync_copy(src_hbm.at[pl.ds(base, PG)], src_pg)
        pltpu.sync_copy(dst_hbm.at[pl.ds(base, PG)], dst_pg)

        def chunk(j, carry2):
            desc = pltpu.async_copy(y_hbm.at[src_pg.at[j]], rows_a, gsem_a)
            count(j)
            desc.wait()
            pltpu.sync_copy(rows_a, agg_sh.at[dst_pg.at[j]], add=True)
            return carry2

        lax.fori_loop(0, PG, chunk, 0)
        return carry

    lax.fori_loop(0, npages, page, 0)

    for t in range(CROWS // 128):
        pltpu.async_copy(cnt_loc.at[pl.ds(t * 128, 128)],
                         cnt_sh.at[iota_v.at[t]], csem, add=True)
    for t in range(CROWS // 128):
        pltpu.make_async_copy(cnt_loc.at[pl.ds(t * 128, 128)],
                              cnt_sh.at[iota_v.at[t]], csem).wait()
    plsc.subcore_barrier()

    pltpu.sync_copy(agg_sh.at[pl.ds(s * ROWS_PER_TILE, ROWS_PER_TILE)],
                    agg_out.at[c, pl.ds(s * ROWS_PER_TILE, ROWS_PER_TILE)])
    pltpu.sync_copy(cnt_sh.at[pl.ds(s * CSTRIPE, CSTRIPE)],
                    cnt_out.at[c, pl.ds(s * CSTRIPE, CSTRIPE)])


def _sc_scatter(y, src_r, dst_r, zrow, zcnt, iota_r):
    mesh = plsc.VectorSubcoreMesh(core_axis_name="c", subcore_axis_name="s")
    fn = pl.kernel(
        _sc_body,
        out_type=(
            jax.ShapeDtypeStruct((NC, NSH, D), jnp.float32),
            jax.ShapeDtypeStruct((NC, CROWS, 16), jnp.float32),
        ),
        mesh=mesh,
        scratch_types=[
            pltpu.VMEM((PG, K), jnp.int32),
            pltpu.VMEM((PG, K), jnp.int32),
            pltpu.VMEM((K, D), jnp.float32),
            pltpu.VMEM((CROWS, 16), jnp.float32),
            pltpu.VMEM((CROWS // 128, 128), jnp.int32),
            pltpu.VMEM_SHARED((NSH, D), jnp.float32),
            pltpu.VMEM_SHARED((CROWS, 16), jnp.float32),
            pltpu.SemaphoreType.DMA,
            pltpu.SemaphoreType.DMA,
        ],
        compiler_params=pltpu.CompilerParams(use_tc_tiling_on_sc=False,
                                             needs_layout_passes=False),
    )
    return fn(y, src_r, dst_r, zrow, zcnt, iota_r)


def _gru_body(y_ref, part_ref, cnt_ref, wih_ref, whh_ref, bih_ref, bhh_ref,
              out_ref):
    y = y_ref[...]
    agg = part_ref[0] + part_ref[1] + y
    cnt = cnt_ref[0] + cnt_ref[1] + 1.0
    h = agg / cnt

    w_ih = wih_ref[...]
    w_hh = whh_ref[...]
    b_ih = bih_ref[...]
    b_hh = bhh_ref[...]

    dn = (((1,), (1,)), ((), ()))
    gi1 = lax.dot_general(h, w_ih, dn, preferred_element_type=jnp.float32) + b_ih
    z1 = jax.nn.sigmoid(gi1[:, 128:256] + b_hh[:, 128:256])
    r1 = jax.nn.sigmoid(gi1[:, 0:128] + b_hh[:, 0:128])
    n1 = jnp.tanh(gi1[:, 256:384] + r1 * b_hh[:, 256:384])
    h1 = (1.0 - z1) * n1
    gi2 = lax.dot_general(y, w_ih, dn, preferred_element_type=jnp.float32) + b_ih
    gh2 = lax.dot_general(h1, w_hh, dn, preferred_element_type=jnp.float32) + b_hh
    r2 = jax.nn.sigmoid(gi2[:, 0:128] + gh2[:, 0:128])
    z2 = jax.nn.sigmoid(gi2[:, 128:256] + gh2[:, 128:256])
    n2 = jnp.tanh(gi2[:, 256:384] + r2 * gh2[:, 256:384])
    out_ref[...] = (1.0 - z2) * n2 + z2 * h1


def _mean_gru(y, part, cntp, w_ih, w_hh, b_ih2, b_hh2):
    blk = 1024
    grid = -(-N // blk)
    return pl.pallas_call(
        _gru_body,
        grid=(grid,),
        in_specs=[
            pl.BlockSpec((blk, D), lambda i: (i, 0)),
            pl.BlockSpec((NC, blk, D), lambda i: (0, i, 0)),
            pl.BlockSpec((NC, blk, 1), lambda i: (0, i, 0)),
            pl.BlockSpec((3 * D, D), lambda i: (0, 0)),
            pl.BlockSpec((3 * D, D), lambda i: (0, 0)),
            pl.BlockSpec((1, 3 * D), lambda i: (0, 0)),
            pl.BlockSpec((1, 3 * D), lambda i: (0, 0)),
        ],
        out_specs=pl.BlockSpec((blk, D), lambda i: (i, 0)),
        out_shape=jax.ShapeDtypeStruct((N, D), jnp.float32),
    )(y, part, cntp, w_ih, w_hh, b_ih2, b_hh2)


def kernel(x, edge_index, W, b, W_ih, W_hh, b_ih, b_hh):
    src = edge_index[0].astype(jnp.int32)
    dst = edge_index[1].astype(jnp.int32)
    pad = EPAD - E
    pad_src = jnp.arange(pad, dtype=jnp.int32) % N
    src_r = jnp.concatenate([src, pad_src]).reshape(TOT_CHUNKS, K)
    pad_dst = N + (jnp.arange(pad, dtype=jnp.int32) % (NSH - N))
    dst_r = jnp.concatenate([dst, pad_dst]).reshape(TOT_CHUNKS, K)

    zrow = jnp.zeros((ROWS_PER_TILE, D), jnp.float32)
    zcnt = jnp.zeros((CROWS, 16), jnp.float32)
    iota_r = jnp.arange(CROWS, dtype=jnp.int32).reshape(CROWS // 128, 128)

    y = _linear(x, W, b.reshape(1, D))
    part, cntp = _sc_scatter(y, src_r, dst_r, zrow, zcnt, iota_r)
    cntp = cntp.reshape(NC, NSH, 1)
    return _mean_gru(y, part, cntp, W_ih, W_hh,
                     b_ih.reshape(1, 3 * D), b_hh.reshape(1, 3 * D))


# speedup vs baseline: 2.3304x; 1.1340x over previous
import functools

import jax
import jax.numpy as jnp
from jax import lax
from jax.experimental import pallas as pl
from jax.experimental.pallas import tpu as pltpu
from jax.experimental.pallas import tpu_sc as plsc

N = 10000
D = 128
E = 320000

NC = 2
NS = 16
NW = NC * NS

K = 128
CH0 = 80
CH1 = 80
PG = 16
TOT_CHUNKS = NS * (CH0 + CH1)
EPAD = TOT_CHUNKS * K
CORE1_ROW = NS * CH0

ROWS_PER_TILE = 640
NSH = ROWS_PER_TILE * NS
CROWS = NSH // 16
CSTRIPE = CROWS // NS


def _linear_body(x_ref, w_ref, b_ref, y_ref):
    y_ref[...] = lax.dot_general(
        x_ref[...], w_ref[...], (((1,), (1,)), ((), ())),
        preferred_element_type=jnp.float32) + b_ref[...]


def _linear(x, w, b2):
    blk = 1000
    return pl.pallas_call(
        _linear_body,
        grid=(N // blk,),
        in_specs=[
            pl.BlockSpec((blk, D), lambda i: (i, 0)),
            pl.BlockSpec((D, D), lambda i: (0, 0)),
            pl.BlockSpec((1, D), lambda i: (0, 0)),
        ],
        out_specs=pl.BlockSpec((blk, D), lambda i: (i, 0)),
        out_shape=jax.ShapeDtypeStruct((N, D), jnp.float32),
    )(x, w, b2)


def _sc_body(y_hbm, src_hbm, dst_hbm, zrow_hbm, zcnt_hbm, iota_hbm,
             agg_out, cnt_out,
             src_pg, dst_pg, rows_a, rows_b, cnt_loc, iota_v, agg_sh, cnt_sh,
             gsem_a, gsem_b, ssem_a, ssem_b, csem):
    c = lax.axis_index("c")
    s = lax.axis_index("s")
    pltpu.sync_copy(zrow_hbm, agg_sh.at[pl.ds(s * ROWS_PER_TILE, ROWS_PER_TILE)])
    pltpu.sync_copy(zcnt_hbm.at[pl.ds(0, CSTRIPE)],
                    cnt_sh.at[pl.ds(s * CSTRIPE, CSTRIPE)])
    pltpu.sync_copy(zcnt_hbm, cnt_loc)
    pltpu.sync_copy(iota_hbm, iota_v)
    plsc.subcore_barrier()

    ones16 = jnp.ones((16,), jnp.float32)

    row0 = jnp.where(c == 0, s * CH0, CORE1_ROW + s * CH1)
    npages = jnp.where(c == 0, CH0 // PG, CH1 // PG)

    def count(j):
        def cbody(i, carry):
            d = dst_pg[j, pl.ds(i * 16, 16)]
            r = lax.shift_right_logical(d, 4)
            col = lax.bitwise_and(d, 15)
            plsc.addupdate_scatter(cnt_loc, [r, col], ones16)
            return carry

        lax.fori_loop(0, K // 16, cbody, 0)

    def g_start(buf, sem, j):
        pltpu.async_copy(y_hbm.at[src_pg.at[j]], buf, sem)

    def g_wait(buf, sem, j):
        pltpu.make_async_copy(y_hbm.at[src_pg.at[j]], buf, sem).wait()

    def s_start(buf, sem, j):
        pltpu.async_copy(buf, agg_sh.at[dst_pg.at[j]], sem, add=True)

    def s_wait(buf, sem, j):
        pltpu.make_async_copy(buf, agg_sh.at[dst_pg.at[j]], sem).wait()

    def page(p, carry):
        base = row0 + p * PG
        pltpu.sync_copy(src_hbm.at[pl.ds(base, PG)], src_pg)
        pltpu.sync_copy(dst_hbm.at[pl.ds(base, PG)], dst_pg)
        g_start(rows_a, gsem_a, 0)

        def pairb(i, carry2):
            j = 2 * i

            @pl.when(i > 0)
            def _():
                s_wait(rows_b, ssem_b, j - 1)

            g_start(rows_b, gsem_b, j + 1)
            count(j)
            g_wait(rows_a, gsem_a, j)
            s_start(rows_a, ssem_a, j)
            count(j + 1)
            g_wait(rows_b, gsem_b, j + 1)
            s_start(rows_b, ssem_b, j + 1)
            s_wait(rows_a, ssem_a, j)

            @pl.when(i < PG // 2 - 1)
            def _():
                g_start(rows_a, gsem_a, j + 2)

            return carry2

        lax.fori_loop(0, PG // 2, pairb, 0)
        s_wait(rows_b, ssem_b, PG - 1)
        return carry

    lax.fori_loop(0, npages, page, 0)

    for t in range(CROWS // 128):
        pltpu.async_copy(cnt_loc.at[pl.ds(t * 128, 128)],
                         cnt_sh.at[iota_v.at[t]], csem, add=True)
    for t in range(CROWS // 128):
        pltpu.make_async_copy(cnt_loc.at[pl.ds(t * 128, 128)],
                              cnt_sh.at[iota_v.at[t]], csem).wait()
    plsc.subcore_barrier()

    pltpu.sync_copy(agg_sh.at[pl.ds(s * ROWS_PER_TILE, ROWS_PER_TILE)],
                    agg_out.at[c, pl.ds(s * ROWS_PER_TILE, ROWS_PER_TILE)])
    pltpu.sync_copy(cnt_sh.at[pl.ds(s * CSTRIPE, CSTRIPE)],
                    cnt_out.at[c, pl.ds(s * CSTRIPE, CSTRIPE)])


def _sc_scatter(y, src_r, dst_r, zrow, zcnt, iota_r):
    mesh = plsc.VectorSubcoreMesh(core_axis_name="c", subcore_axis_name="s")
    fn = pl.kernel(
        _sc_body,
        out_type=(
            jax.ShapeDtypeStruct((NC, NSH, D), jnp.float32),
            jax.ShapeDtypeStruct((NC, CROWS, 16), jnp.float32),
        ),
        mesh=mesh,
        scratch_types=[
            pltpu.VMEM((PG, K), jnp.int32),
            pltpu.VMEM((PG, K), jnp.int32),
            pltpu.VMEM((K, D), jnp.float32),
            pltpu.VMEM((K, D), jnp.float32),
            pltpu.VMEM((CROWS, 16), jnp.float32),
            pltpu.VMEM((CROWS // 128, 128), jnp.int32),
            pltpu.VMEM_SHARED((NSH, D), jnp.float32),
            pltpu.VMEM_SHARED((CROWS, 16), jnp.float32),
            pltpu.SemaphoreType.DMA,
            pltpu.SemaphoreType.DMA,
            pltpu.SemaphoreType.DMA,
            pltpu.SemaphoreType.DMA,
            pltpu.SemaphoreType.DMA,
        ],
        compiler_params=pltpu.CompilerParams(use_tc_tiling_on_sc=False,
                                             needs_layout_passes=False),
    )
    return fn(y, src_r, dst_r, zrow, zcnt, iota_r)


def _gru_body(y_ref, part_ref, cnt_ref, wih_ref, whh_ref, bih_ref, bhh_ref,
              out_ref):
    y = y_ref[...]
    agg = part_ref[0] + part_ref[1] + y
    cnt = cnt_ref[0] + cnt_ref[1] + 1.0
    h = agg / cnt

    w_ih = wih_ref[...]
    w_hh = whh_ref[...]
    b_ih = bih_ref[...]
    b_hh = bhh_ref[...]

    dn = (((1,), (1,)), ((), ()))
    gi1 = lax.dot_general(h, w_ih, dn, preferred_element_type=jnp.float32) + b_ih
    z1 = jax.nn.sigmoid(gi1[:, 128:256] + b_hh[:, 128:256])
    r1 = jax.nn.sigmoid(gi1[:, 0:128] + b_hh[:, 0:128])
    n1 = jnp.tanh(gi1[:, 256:384] + r1 * b_hh[:, 256:384])
    h1 = (1.0 - z1) * n1
    gi2 = lax.dot_general(y, w_ih, dn, preferred_element_type=jnp.float32) + b_ih
    gh2 = lax.dot_general(h1, w_hh, dn, preferred_element_type=jnp.float32) + b_hh
    r2 = jax.nn.sigmoid(gi2[:, 0:128] + gh2[:, 0:128])
    z2 = jax.nn.sigmoid(gi2[:, 128:256] + gh2[:, 128:256])
    n2 = jnp.tanh(gi2[:, 256:384] + r2 * gh2[:, 256:384])
    out_ref[...] = (1.0 - z2) * n2 + z2 * h1


def _mean_gru(y, part, cntp, w_ih, w_hh, b_ih2, b_hh2):
    blk = 1024
    grid = -(-N // blk)
    return pl.pallas_call(
        _gru_body,
        grid=(grid,),
        in_specs=[
            pl.BlockSpec((blk, D), lambda i: (i, 0)),
            pl.BlockSpec((NC, blk, D), lambda i: (0, i, 0)),
            pl.BlockSpec((NC, blk, 1), lambda i: (0, i, 0)),
            pl.BlockSpec((3 * D, D), lambda i: (0, 0)),
            pl.BlockSpec((3 * D, D), lambda i: (0, 0)),
            pl.BlockSpec((1, 3 * D), lambda i: (0, 0)),
            pl.BlockSpec((1, 3 * D), lambda i: (0, 0)),
        ],
        out_specs=pl.BlockSpec((blk, D), lambda i: (i, 0)),
        out_shape=jax.ShapeDtypeStruct((N, D), jnp.float32),
    )(y, part, cntp, w_ih, w_hh, b_ih2, b_hh2)


def kernel(x, edge_index, W, b, W_ih, W_hh, b_ih, b_hh):
    src = edge_index[0].astype(jnp.int32)
    dst = edge_index[1].astype(jnp.int32)
    pad = EPAD - E
    pad_src = jnp.arange(pad, dtype=jnp.int32) % N
    src_r = jnp.concatenate([src, pad_src]).reshape(TOT_CHUNKS, K)
    pad_dst = N + (jnp.arange(pad, dtype=jnp.int32) % (NSH - N))
    dst_r = jnp.concatenate([dst, pad_dst]).reshape(TOT_CHUNKS, K)

    zrow = jnp.zeros((ROWS_PER_TILE, D), jnp.float32)
    zcnt = jnp.zeros((CROWS, 16), jnp.float32)
    iota_r = jnp.arange(CROWS, dtype=jnp.int32).reshape(CROWS // 128, 128)

    y = _linear(x, W, b.reshape(1, D))
    part, cntp = _sc_scatter(y, src_r, dst_r, zrow, zcnt, iota_r)
    cntp = cntp.reshape(NC, NSH, 1)
    return _mean_gru(y, part, cntp, W_ih, W_hh,
                     b_ih.reshape(1, 3 * D), b_hh.reshape(1, 3 * D))


# speedup vs baseline: 2.3937x; 1.0271x over previous
import functools

import jax
import jax.numpy as jnp
from jax import lax
from jax.experimental import pallas as pl
from jax.experimental.pallas import tpu as pltpu
from jax.experimental.pallas import tpu_sc as plsc

N = 10000
D = 128
E = 320000

NC = 2
NS = 16
NW = NC * NS

K = 128
CH0 = 80
CH1 = 80
PG = 16
TOT_CHUNKS = NS * (CH0 + CH1)
EPAD = TOT_CHUNKS * K
CORE1_ROW = NS * CH0

ROWS_PER_TILE = 640
NSH = ROWS_PER_TILE * NS
CROWS = NSH // 16
CSTRIPE = CROWS // NS


def _linear_body(x_ref, w_ref, b_ref, y_ref):
    y_ref[...] = lax.dot_general(
        x_ref[...], w_ref[...], (((1,), (1,)), ((), ())),
        preferred_element_type=jnp.float32) + b_ref[...]


def _linear(x, w, b2):
    blk = 1000
    return pl.pallas_call(
        _linear_body,
        grid=(N // blk,),
        in_specs=[
            pl.BlockSpec((blk, D), lambda i: (i, 0)),
            pl.BlockSpec((D, D), lambda i: (0, 0)),
            pl.BlockSpec((1, D), lambda i: (0, 0)),
        ],
        out_specs=pl.BlockSpec((blk, D), lambda i: (i, 0)),
        out_shape=jax.ShapeDtypeStruct((N, D), jnp.float32),
    )(x, w, b2)


def _sc_body(y_hbm, src_hbm, dst_hbm, zrow_hbm, zcnt_hbm, iota_hbm,
             agg_out, cnt_out,
             src_pg, dst_pg, rows_a, rows_b, cnt_loc, iota_v, agg_sh, cnt_sh,
             gsem_a, gsem_b, ssem_a, ssem_b, csem):
    c = lax.axis_index("c")
    s = lax.axis_index("s")
    pltpu.sync_copy(zrow_hbm, agg_sh.at[pl.ds(s * ROWS_PER_TILE, ROWS_PER_TILE)])
    pltpu.sync_copy(zcnt_hbm.at[pl.ds(0, CSTRIPE)],
                    cnt_sh.at[pl.ds(s * CSTRIPE, CSTRIPE)])
    pltpu.sync_copy(zcnt_hbm, cnt_loc)
    pltpu.sync_copy(iota_hbm, iota_v)
    plsc.subcore_barrier()

    ones16 = jnp.ones((16,), jnp.float32)

    row0 = jnp.where(c == 0, s * CH0, CORE1_ROW + s * CH1)
    npages = jnp.where(c == 0, CH0 // PG, CH1 // PG)

    def count(j):
        def cbody(i, carry):
            d = dst_pg[j, pl.ds(i * 16, 16)]
            r = lax.shift_right_logical(d, 4)
            col = lax.bitwise_and(d, 15)
            plsc.addupdate_scatter(cnt_loc, [r, col], ones16)
            return carry

        lax.fori_loop(0, K // 16, cbody, 0)

    def g_start(buf, sem, j):
        pltpu.async_copy(y_hbm.at[src_pg.at[j]], buf, sem)

    def g_wait(buf, sem, j):
        pltpu.make_async_copy(y_hbm.at[src_pg.at[j]], buf, sem).wait()

    def s_start(buf, sem, j):
        pltpu.async_copy(buf, agg_sh.at[dst_pg.at[j]], sem, add=True)

    def s_wait(buf, sem, j):
        pltpu.make_async_copy(buf, agg_sh.at[dst_pg.at[j]], sem).wait()

    def page(p, carry):
        base = row0 + p * PG
        pltpu.sync_copy(src_hbm.at[pl.ds(base, PG)], src_pg)
        pltpu.sync_copy(dst_hbm.at[pl.ds(base, PG)], dst_pg)
        g_start(rows_a, gsem_a, 0)

        def pairb(i, carry2):
            j = 2 * i

            @pl.when(i > 0)
            def _():
                s_wait(rows_b, ssem_b, j - 1)

            g_start(rows_b, gsem_b, j + 1)
            count(j)
            g_wait(rows_a, gsem_a, j)
            s_start(rows_a, ssem_a, j)
            count(j + 1)
            g_wait(rows_b, gsem_b, j + 1)
            s_start(rows_b, ssem_b, j + 1)
            s_wait(rows_a, ssem_a, j)

            @pl.when(i < PG // 2 - 1)
            def _():
                g_start(rows_a, gsem_a, j + 2)

            return carry2

        lax.fori_loop(0, PG // 2, pairb, 0)
        s_wait(rows_b, ssem_b, PG - 1)
        return carry

    lax.fori_loop(0, npages, page, 0)

    for t in range(CROWS // 128):
        pltpu.async_copy(cnt_loc.at[pl.ds(t * 128, 128)],
                         cnt_sh.at[iota_v.at[t]], csem, add=True)
    for t in range(CROWS // 128):
        pltpu.make_async_copy(cnt_loc.at[pl.ds(t * 128, 128)],
                              cnt_sh.at[iota_v.at[t]], csem).wait()
    plsc.subcore_barrier()

    pltpu.sync_copy(agg_sh.at[pl.ds(s * ROWS_PER_TILE, ROWS_PER_TILE)],
                    agg_out.at[c, pl.ds(s * ROWS_PER_TILE, ROWS_PER_TILE)])
    pltpu.sync_copy(cnt_sh.at[pl.ds(s * CSTRIPE, CSTRIPE)],
                    cnt_out.at[c, pl.ds(s * CSTRIPE, CSTRIPE)])


def _sc_scatter(y, src_r, dst_r, zrow, zcnt, iota_r):
    mesh = plsc.VectorSubcoreMesh(core_axis_name="c", subcore_axis_name="s")
    fn = pl.kernel(
        _sc_body,
        out_type=(
            jax.ShapeDtypeStruct((NC, NSH, D), jnp.float32),
            jax.ShapeDtypeStruct((NC, CROWS, 16), jnp.float32),
        ),
        mesh=mesh,
        scratch_types=[
            pltpu.VMEM((PG, K), jnp.int32),
            pltpu.VMEM((PG, K), jnp.int32),
            pltpu.VMEM((K, D), jnp.float32),
            pltpu.VMEM((K, D), jnp.float32),
            pltpu.VMEM((CROWS, 16), jnp.float32),
            pltpu.VMEM((CROWS // 128, 128), jnp.int32),
            pltpu.VMEM_SHARED((NSH, D), jnp.float32),
            pltpu.VMEM_SHARED((CROWS, 16), jnp.float32),
            pltpu.SemaphoreType.DMA,
            pltpu.SemaphoreType.DMA,
            pltpu.SemaphoreType.DMA,
            pltpu.SemaphoreType.DMA,
            pltpu.SemaphoreType.DMA,
        ],
        compiler_params=pltpu.CompilerParams(use_tc_tiling_on_sc=False,
                                             needs_layout_passes=False),
    )
    return fn(y, src_r, dst_r, zrow, zcnt, iota_r)


def _gru_body(x_ref, y_ref, part_ref, cnt_ref, w_ref, b_ref,
              wih_ref, whh_ref, bih_ref, bhh_ref, out_ref):
    y = y_ref[...]
    aggx = part_ref[0] + part_ref[1] + x_ref[...]
    cnt = cnt_ref[0] + cnt_ref[1] + 1.0
    dn = (((1,), (1,)), ((), ()))
    h = lax.dot_general(aggx / cnt, w_ref[...], dn,
                        preferred_element_type=jnp.float32) + b_ref[...]

    w_ih = wih_ref[...]
    w_hh = whh_ref[...]
    b_ih = bih_ref[...]
    b_hh = bhh_ref[...]

    gi1 = lax.dot_general(h, w_ih, dn, preferred_element_type=jnp.float32) + b_ih
    z1 = jax.nn.sigmoid(gi1[:, 128:256] + b_hh[:, 128:256])
    r1 = jax.nn.sigmoid(gi1[:, 0:128] + b_hh[:, 0:128])
    n1 = jnp.tanh(gi1[:, 256:384] + r1 * b_hh[:, 256:384])
    h1 = (1.0 - z1) * n1
    gi2 = lax.dot_general(y, w_ih, dn, preferred_element_type=jnp.float32) + b_ih
    gh2 = lax.dot_general(h1, w_hh, dn, preferred_element_type=jnp.float32) + b_hh
    r2 = jax.nn.sigmoid(gi2[:, 0:128] + gh2[:, 0:128])
    z2 = jax.nn.sigmoid(gi2[:, 128:256] + gh2[:, 128:256])
    n2 = jnp.tanh(gi2[:, 256:384] + r2 * gh2[:, 256:384])
    out_ref[...] = (1.0 - z2) * n2 + z2 * h1


def _mean_gru(x, y, part, cntp, w, b2, w_ih, w_hh, b_ih2, b_hh2):
    blk = 1024
    grid = -(-N // blk)
    return pl.pallas_call(
        _gru_body,
        grid=(grid,),
        in_specs=[
            pl.BlockSpec((blk, D), lambda i: (i, 0)),
            pl.BlockSpec((blk, D), lambda i: (i, 0)),
            pl.BlockSpec((NC, blk, D), lambda i: (0, i, 0)),
            pl.BlockSpec((NC, blk, 1), lambda i: (0, i, 0)),
            pl.BlockSpec((D, D), lambda i: (0, 0)),
            pl.BlockSpec((1, D), lambda i: (0, 0)),
            pl.BlockSpec((3 * D, D), lambda i: (0, 0)),
            pl.BlockSpec((3 * D, D), lambda i: (0, 0)),
            pl.BlockSpec((1, 3 * D), lambda i: (0, 0)),
            pl.BlockSpec((1, 3 * D), lambda i: (0, 0)),
        ],
        out_specs=pl.BlockSpec((blk, D), lambda i: (i, 0)),
        out_shape=jax.ShapeDtypeStruct((N, D), jnp.float32),
    )(x, y, part, cntp, w, b2, w_ih, w_hh, b_ih2, b_hh2)


def kernel(x, edge_index, W, b, W_ih, W_hh, b_ih, b_hh):
    src = edge_index[0].astype(jnp.int32)
    dst = edge_index[1].astype(jnp.int32)
    pad = EPAD - E
    pad_src = jnp.arange(pad, dtype=jnp.int32) % N
    src_r = jnp.concatenate([src, pad_src]).reshape(TOT_CHUNKS, K)
    pad_dst = N + (jnp.arange(pad, dtype=jnp.int32) % (NSH - N))
    dst_r = jnp.concatenate([dst, pad_dst]).reshape(TOT_CHUNKS, K)

    zrow = jnp.zeros((ROWS_PER_TILE, D), jnp.float32)
    zcnt = jnp.zeros((CROWS, 16), jnp.float32)
    iota_r = jnp.arange(CROWS, dtype=jnp.int32).reshape(CROWS // 128, 128)

    part, cntp = _sc_scatter(x, src_r, dst_r, zrow, zcnt, iota_r)
    y = _linear(x, W, b.reshape(1, D))
    cntp = cntp.reshape(NC, NSH, 1)
    return _mean_gru(x, y, part, cntp, W, b.reshape(1, D), W_ih, W_hh,
                     b_ih.reshape(1, 3 * D), b_hh.reshape(1, 3 * D))
